# baseline (speedup 1.0000x reference)
import jax
import jax.numpy as jnp
from jax import lax
from jax.experimental import pallas as pl
from jax.experimental.pallas import tpu as pltpu
from jax.experimental.pallas import tpu_sc as plsc
import functools

N_NODES = 10000
NUM_REL = 16
DIM = 128
E_TOTAL = 160000
BS = 2
NEG = 8
P = BS * NEG
R2 = 2 * NUM_REL
L = 16
NSUB = 32
CH = 5120
E_PAD = NSUB * CH
NVEC = CH // L
HIST = P * R2 * L


def _sc_count_body(src_hbm, dst_hbm, et_hbm, pair_hbm, out_hbm, scr_hbm,
                   s_v, d_v, t_v, pair_v, cnt_v, scr_v, scr2_v):
    wid = lax.axis_index("s") * 2 + lax.axis_index("c")
    base = wid * CH
    pltpu.sync_copy(src_hbm.at[pl.ds(base, CH)], s_v)
    pltpu.sync_copy(dst_hbm.at[pl.ds(base, CH)], d_v)
    pltpu.sync_copy(et_hbm.at[pl.ds(base, CH)], t_v)
    pltpu.sync_copy(pair_hbm, pair_v)

    def _zero(k, c):
        cnt_v[pl.ds(k * L, L)] = jnp.zeros((L,), jnp.float32)
        return c
    lax.fori_loop(0, HIST // L, _zero, 0)

    def hsld(b):
        return pair_v[pl.ds(b * L, L)]

    def tsld(p):
        return pair_v[pl.ds((BS + p) * L, L)]

    def _screen(v, c):
        off = v * L
        sv = s_v[pl.ds(off, L)]
        dv = d_v[pl.ds(off, L)]
        m_any = ((sv == hsld(0)) | (sv == hsld(1)) |
                 (dv == hsld(0)) | (dv == hsld(1)))
        scr_v[pl.ds(off, L)] = jnp.where(m_any, jnp.int32(1), jnp.int32(0))
        return c
    lax.fori_loop(0, NVEC, _screen, 0)

    pltpu.sync_copy(scr_v, scr_hbm.at[wid])
    pltpu.sync_copy(scr_hbm.at[wid], scr2_v)

    def _step(v, c):
        off = v * L
        w = scr2_v[pl.ds(off, L)]
        hit = w[0]
        for i in range(1, L):
            hit = hit | w[i]

        @pl.when(hit > 0)
        def _count():
            sv = s_v[pl.ds(off, L)]
            dv = d_v[pl.ds(off, L)]
            tvv = t_v[pl.ds(off, L)]
            m1 = []
            m2 = []
            for b in range(BS):
                mf = sv == hsld(b)
                mi = dv == hsld(b)
                for j in range(NEG):
                    p = b * NEG + j
                    m1.append(jnp.logical_and(mf, dv == tsld(p)))
                    m2.append(jnp.logical_and(mi, sv == tsld(p)))
            for r in range(NUM_REL):
                eq = tvv == r
                for p in range(P):
                    v1 = jnp.where(jnp.logical_and(m1[p], eq), 1.0, 0.0)
                    plsc.addupdate(cnt_v.at[pl.ds((p * R2 + r) * L, L)], v1)
                    v2 = jnp.where(jnp.logical_and(m2[p], eq), 1.0, 0.0)
                    plsc.addupdate(
                        cnt_v.at[pl.ds((p * R2 + NUM_REL + r) * L, L)], v2)
        return c
    lax.fori_loop(0, NVEC, _step, 0)

    pltpu.sync_copy(cnt_v, out_hbm.at[wid])


_sc_count = functools.partial(
    pl.kernel,
    out_type=(jax.ShapeDtypeStruct((NSUB, HIST), jnp.float32),
              jax.ShapeDtypeStruct((NSUB, CH), jnp.int32)),
    mesh=plsc.VectorSubcoreMesh(core_axis_name="c", subcore_axis_name="s"),
    scratch_types=[
        pltpu.VMEM((CH,), jnp.int32),
        pltpu.VMEM((CH,), jnp.int32),
        pltpu.VMEM((CH,), jnp.int32),
        pltpu.VMEM(((BS + P) * L,), jnp.int32),
        pltpu.VMEM((HIST,), jnp.float32),
        pltpu.VMEM((CH,), jnp.int32),
        pltpu.VMEM((CH,), jnp.int32),
    ],
)(_sc_count_body)


def _tc_tail_body(cnt_ref, hv_ref, tv_ref, rv_ref,
                  rq_ref, w_ref, b_ref, wm1_ref, bm1_ref, wm2_ref, bm2_ref,
                  out_ref):
    X = cnt_ref[...]
    row_i = jax.lax.broadcasted_iota(jnp.int32, (P, NSUB * P), 1)
    p_i = jax.lax.broadcasted_iota(jnp.int32, (P, NSUB * P), 0)
    sel = (lax.rem(row_i, P) == p_i).astype(jnp.float32)
    cnt_pl = sel @ X
    col_i = jax.lax.broadcasted_iota(jnp.int32, (R2 * L, R2), 0)
    r_i = jax.lax.broadcasted_iota(jnp.int32, (R2 * L, R2), 1)
    red = (lax.div(col_i, L) == r_i).astype(jnp.float32)
    cnt = cnt_pl @ red

    rq = rq_ref[...]
    S = cnt @ rq

    r_iota = jax.lax.broadcasted_iota(jnp.int32, (P, R2), 1)
    oh_r = (rv_ref[...] == r_iota).astype(jnp.float32)
    q = oh_r @ rq

    is_head = (tv_ref[...] == hv_ref[...]).astype(jnp.float32)
    agg = q * (S + is_head)
    hin = is_head * q
    h1 = jnp.maximum(hin @ w_ref[:DIM] + agg @ w_ref[DIM:] + b_ref[...], 0.0)
    hm = jnp.maximum(h1 @ wm1_ref[:DIM] + q @ wm1_ref[DIM:] + bm1_ref[...], 0.0)
    out_ref[...] = hm @ wm2_ref[...] + bm2_ref[...]


def kernel(edge_index, edge_type, rel_query, h_index, t_index, r_index,
           W, b, Wm1, bm1, Wm2, bm2):
    npad = E_PAD - E_TOTAL
    pad_i = jnp.full((npad,), -1, jnp.int32)
    pad_t = jnp.zeros((npad,), jnp.int32)
    src = jnp.concatenate([edge_index[0], pad_i])
    dst = jnp.concatenate([edge_index[1], pad_i])
    et = jnp.concatenate([edge_type, pad_t])
    hv = h_index.reshape(P)
    tv = t_index.reshape(P)
    pair = jnp.concatenate([jnp.repeat(h_index[:, 0], L),
                            jnp.repeat(tv, L)])

    cnt_all, _ = _sc_count(src, dst, et, pair)
    cnt2 = cnt_all.reshape(NSUB * P, R2 * L)

    whole = lambda shape: pl.BlockSpec(shape, lambda: tuple(0 for _ in shape))
    out = pl.pallas_call(
        _tc_tail_body,
        in_specs=[
            whole((NSUB * P, R2 * L)),
            whole((P, 1)), whole((P, 1)), whole((P, 1)),
            whole((R2, DIM)),
            whole((2 * DIM, DIM)), whole((1, DIM)),
            whole((2 * DIM, 2 * DIM)), whole((1, 2 * DIM)),
            whole((2 * DIM, 1)), whole((1, 1)),
        ],
        out_specs=whole((P, 1)),
        out_shape=jax.ShapeDtypeStruct((P, 1), jnp.float32),
    )(cnt2, hv.reshape(P, 1), tv.reshape(P, 1), r_index.reshape(P, 1),
      rel_query, W, b.reshape(1, DIM), Wm1, bm1.reshape(1, 2 * DIM),
      Wm2, bm2.reshape(1, 1))
    return out[:, 0].reshape(BS, NEG)


# speedup vs baseline: 1.1018x; 1.1018x over previous
import jax
import jax.numpy as jnp
from jax import lax
from jax.experimental import pallas as pl
from jax.experimental.pallas import tpu as pltpu
from jax.experimental.pallas import tpu_sc as plsc
import functools

N_NODES = 10000
NUM_REL = 16
DIM = 128
E_TOTAL = 160000
BS = 2
NEG = 8
P = BS * NEG
R2 = 2 * NUM_REL
L = 16
NSUB = 32
CH = 5120
E_PAD = NSUB * CH
NVEC = CH // L
HIST = P * R2 * L
G = 8
NGRP = NVEC // G
SCR = (NVEC + NGRP) * L


def _sc_count_body(src_hbm, dst_hbm, et_hbm, pair_hbm, out_hbm, scr_hbm,
                   s_v, d_v, t_v, pair_v, cnt_v, scr_v, scr2_v, sem):
    wid = lax.axis_index("s") * 2 + lax.axis_index("c")
    base = wid * CH
    pltpu.sync_copy(src_hbm.at[pl.ds(base, CH)], s_v)
    pltpu.sync_copy(dst_hbm.at[pl.ds(base, CH)], d_v)
    pltpu.sync_copy(et_hbm.at[pl.ds(base, CH)], t_v)
    pltpu.sync_copy(pair_hbm, pair_v)

    def _zero(k, c):
        cnt_v[pl.ds(k * L, L)] = jnp.zeros((L,), jnp.float32)
        return c
    lax.fori_loop(0, HIST // L, _zero, 0)

    def hsld(b):
        return pair_v[pl.ds(b * L, L)]

    def tsld(p):
        return pair_v[pl.ds((BS + p) * L, L)]

    def _screen(g, c):
        off = g * G * L
        acc = jnp.zeros((L,), jnp.int32)
        for j in range(G):
            sv = s_v[pl.ds(off + j * L, L)]
            dv = d_v[pl.ds(off + j * L, L)]
            m_any = ((sv == hsld(0)) | (sv == hsld(1)) |
                     (dv == hsld(0)) | (dv == hsld(1)))
            w = jnp.where(m_any, jnp.int32(1), jnp.int32(0))
            scr_v[pl.ds(off + j * L, L)] = w
            acc = acc | w
        scr_v[pl.ds((NVEC + g) * L, L)] = acc
        return c
    lax.fori_loop(0, NGRP, _screen, 0)

    pltpu.sync_copy(scr_v, scr_hbm.at[wid])
    pltpu.sync_copy(scr_hbm.at[wid], scr2_v)

    def _count_vec(off):
        sv = s_v[pl.ds(off, L)]
        dv = d_v[pl.ds(off, L)]
        tvv = t_v[pl.ds(off, L)]
        m1 = []
        m2 = []
        for b in range(BS):
            mf = sv == hsld(b)
            mi = dv == hsld(b)
            for j in range(NEG):
                p = b * NEG + j
                m1.append(jnp.logical_and(mf, dv == tsld(p)))
                m2.append(jnp.logical_and(mi, sv == tsld(p)))
        for r in range(NUM_REL):
            eq = tvv == r
            for p in range(P):
                v1 = jnp.where(jnp.logical_and(m1[p], eq), 1.0, 0.0)
                plsc.addupdate(cnt_v.at[pl.ds((p * R2 + r) * L, L)], v1)
                v2 = jnp.where(jnp.logical_and(m2[p], eq), 1.0, 0.0)
                plsc.addupdate(
                    cnt_v.at[pl.ds((p * R2 + NUM_REL + r) * L, L)], v2)

    def _step(g, c):
        goff = (NVEC + g) * L
        w = scr2_v[pl.ds(goff, L)]
        hit = w[0]
        for i in range(1, L):
            hit = hit | w[i]

        @pl.when(hit > 0)
        def _drill():
            def _dj(j, c2):
                off = (g * G + j) * L
                wv = scr2_v[pl.ds(off, L)]
                h2 = wv[0]
                for i in range(1, L):
                    h2 = h2 | wv[i]

                @pl.when(h2 > 0)
                def _count():
                    _count_vec(off)
                return c2
            lax.fori_loop(0, G, _dj, 0)
        return c
    lax.fori_loop(0, NGRP, _step, 0)

    pltpu.sync_copy(cnt_v, out_hbm.at[wid])


_sc_count = functools.partial(
    pl.kernel,
    out_type=(jax.ShapeDtypeStruct((NSUB, HIST), jnp.float32),
              jax.ShapeDtypeStruct((NSUB, SCR), jnp.int32)),
    mesh=plsc.VectorSubcoreMesh(core_axis_name="c", subcore_axis_name="s"),
    scratch_types=[
        pltpu.VMEM((CH,), jnp.int32),
        pltpu.VMEM((CH,), jnp.int32),
        pltpu.VMEM((CH,), jnp.int32),
        pltpu.VMEM(((BS + P) * L,), jnp.int32),
        pltpu.VMEM((HIST,), jnp.float32),
        pltpu.VMEM((SCR,), jnp.int32),
        pltpu.VMEM((SCR,), jnp.int32),
        pltpu.SemaphoreType.DMA,
    ],
)(_sc_count_body)


def _tc_tail_body(cnt_ref, hv_ref, tv_ref, rv_ref,
                  rq_ref, w_ref, b_ref, wm1_ref, bm1_ref, wm2_ref, bm2_ref,
                  out_ref):
    X = cnt_ref[...]
    row_i = jax.lax.broadcasted_iota(jnp.int32, (P, NSUB * P), 1)
    p_i = jax.lax.broadcasted_iota(jnp.int32, (P, NSUB * P), 0)
    sel = (lax.rem(row_i, P) == p_i).astype(jnp.float32)
    cnt_pl = sel @ X
    col_i = jax.lax.broadcasted_iota(jnp.int32, (R2 * L, R2), 0)
    r_i = jax.lax.broadcasted_iota(jnp.int32, (R2 * L, R2), 1)
    red = (lax.div(col_i, L) == r_i).astype(jnp.float32)
    cnt = cnt_pl @ red

    rq = rq_ref[...]
    S = cnt @ rq

    r_iota = jax.lax.broadcasted_iota(jnp.int32, (P, R2), 1)
    oh_r = (rv_ref[...] == r_iota).astype(jnp.float32)
    q = oh_r @ rq

    is_head = (tv_ref[...] == hv_ref[...]).astype(jnp.float32)
    agg = q * (S + is_head)
    hin = is_head * q
    h1 = jnp.maximum(hin @ w_ref[:DIM] + agg @ w_ref[DIM:] + b_ref[...], 0.0)
    hm = jnp.maximum(h1 @ wm1_ref[:DIM] + q @ wm1_ref[DIM:] + bm1_ref[...], 0.0)
    out_ref[...] = hm @ wm2_ref[...] + bm2_ref[...]


def kernel(edge_index, edge_type, rel_query, h_index, t_index, r_index,
           W, b, Wm1, bm1, Wm2, bm2):
    npad = E_PAD - E_TOTAL
    pad_i = jnp.full((npad,), -1, jnp.int32)
    pad_t = jnp.zeros((npad,), jnp.int32)
    src = jnp.concatenate([edge_index[0], pad_i])
    dst = jnp.concatenate([edge_index[1], pad_i])
    et = jnp.concatenate([edge_type, pad_t])
    hv = h_index.reshape(P)
    tv = t_index.reshape(P)
    pair = jnp.concatenate([jnp.repeat(h_index[:, 0], L),
                            jnp.repeat(tv, L)])

    cnt_all, _ = _sc_count(src, dst, et, pair)
    cnt2 = cnt_all.reshape(NSUB * P, R2 * L)

    whole = lambda shape: pl.BlockSpec(shape, lambda: tuple(0 for _ in shape))
    out = pl.pallas_call(
        _tc_tail_body,
        in_specs=[
            whole((NSUB * P, R2 * L)),
            whole((P, 1)), whole((P, 1)), whole((P, 1)),
            whole((R2, DIM)),
            whole((2 * DIM, DIM)), whole((1, DIM)),
            whole((2 * DIM, 2 * DIM)), whole((1, 2 * DIM)),
            whole((2 * DIM, 1)), whole((1, 1)),
        ],
        out_specs=whole((P, 1)),
        out_shape=jax.ShapeDtypeStruct((P, 1), jnp.float32),
    )(cnt2, hv.reshape(P, 1), tv.reshape(P, 1), r_index.reshape(P, 1),
      rel_query, W, b.reshape(1, DIM), Wm1, bm1.reshape(1, 2 * DIM),
      Wm2, bm2.reshape(1, 1))
    return out[:, 0].reshape(BS, NEG)


# speedup vs baseline: 1.1805x; 1.0714x over previous
import jax
import jax.numpy as jnp
from jax import lax
from jax.experimental import pallas as pl
from jax.experimental.pallas import tpu as pltpu
from jax.experimental.pallas import tpu_sc as plsc
import functools

N_NODES = 10000
NUM_REL = 16
DIM = 128
E_TOTAL = 160000
BS = 2
NEG = 8
P = BS * NEG
R2 = 2 * NUM_REL
L = 16
NSUB = 32
CH = 5120
E_PAD = NSUB * CH
NVEC = CH // L
HIST = P * R2 * L
G = 8
NGRP = NVEC // G
SCR = (NVEC + NGRP) * L


def _sc_count_body(src_hbm, dst_hbm, et_hbm, pair_hbm, out_hbm, scr_hbm,
                   s_v, d_v, t_v, pair_v, cnt_v, scr_v, scr2_v, sem):
    wid = lax.axis_index("s") * 2 + lax.axis_index("c")
    base = wid * CH
    c1 = pltpu.async_copy(src_hbm.at[pl.ds(base, CH)], s_v, sem)
    c2 = pltpu.async_copy(dst_hbm.at[pl.ds(base, CH)], d_v, sem)
    c3 = pltpu.async_copy(et_hbm.at[pl.ds(base, CH)], t_v, sem)
    c4 = pltpu.async_copy(pair_hbm, pair_v, sem)

    zv = jnp.zeros((L,), jnp.float32)

    def _zero(k, c):
        for u in range(8):
            cnt_v[pl.ds((k * 8 + u) * L, L)] = zv
        return c
    lax.fori_loop(0, HIST // (8 * L), _zero, 0)
    c1.wait()
    c2.wait()
    c3.wait()
    c4.wait()

    def hsld(b):
        return pair_v[pl.ds(b * L, L)]

    def tsld(p):
        return pair_v[pl.ds((BS + p) * L, L)]

    def _screen(g, c):
        off = g * G * L
        acc = jnp.zeros((L,), jnp.int32)
        for j in range(G):
            sv = s_v[pl.ds(off + j * L, L)]
            dv = d_v[pl.ds(off + j * L, L)]
            m_any = ((sv == hsld(0)) | (sv == hsld(1)) |
                     (dv == hsld(0)) | (dv == hsld(1)))
            w = jnp.where(m_any, jnp.int32(1), jnp.int32(0))
            scr_v[pl.ds(off + j * L, L)] = w
            acc = acc | w
        scr_v[pl.ds((NVEC + g) * L, L)] = acc
        return c
    lax.fori_loop(0, NGRP, _screen, 0)

    pltpu.sync_copy(scr_v, scr_hbm.at[wid])
    pltpu.sync_copy(scr_hbm.at[wid], scr2_v)

    def _count_vec(off):
        sv = s_v[pl.ds(off, L)]
        dv = d_v[pl.ds(off, L)]
        tvv = t_v[pl.ds(off, L)]
        m1 = []
        m2 = []
        for b in range(BS):
            mf = sv == hsld(b)
            mi = dv == hsld(b)
            for j in range(NEG):
                p = b * NEG + j
                m1.append(jnp.logical_and(mf, dv == tsld(p)))
                m2.append(jnp.logical_and(mi, sv == tsld(p)))
        for r in range(NUM_REL):
            eq = tvv == r
            for p in range(P):
                v1 = jnp.where(jnp.logical_and(m1[p], eq), 1.0, 0.0)
                plsc.addupdate(cnt_v.at[pl.ds((p * R2 + r) * L, L)], v1)
                v2 = jnp.where(jnp.logical_and(m2[p], eq), 1.0, 0.0)
                plsc.addupdate(
                    cnt_v.at[pl.ds((p * R2 + NUM_REL + r) * L, L)], v2)

    def _step(g, c):
        goff = (NVEC + g) * L
        w = scr2_v[pl.ds(goff, L)]
        hit = w[0]
        for i in range(1, L):
            hit = hit | w[i]

        @pl.when(hit > 0)
        def _drill():
            def _dj(j, c2):
                off = (g * G + j) * L
                wv = scr2_v[pl.ds(off, L)]
                h2 = wv[0]
                for i in range(1, L):
                    h2 = h2 | wv[i]

                @pl.when(h2 > 0)
                def _count():
                    _count_vec(off)
                return c2
            lax.fori_loop(0, G, _dj, 0)
        return c
    lax.fori_loop(0, NGRP, _step, 0)

    pltpu.sync_copy(cnt_v, out_hbm.at[wid])


_sc_count = functools.partial(
    pl.kernel,
    out_type=(jax.ShapeDtypeStruct((NSUB, HIST), jnp.float32),
              jax.ShapeDtypeStruct((NSUB, SCR), jnp.int32)),
    mesh=plsc.VectorSubcoreMesh(core_axis_name="c", subcore_axis_name="s"),
    scratch_types=[
        pltpu.VMEM((CH,), jnp.int32),
        pltpu.VMEM((CH,), jnp.int32),
        pltpu.VMEM((CH,), jnp.int32),
        pltpu.VMEM(((BS + P) * L,), jnp.int32),
        pltpu.VMEM((HIST,), jnp.float32),
        pltpu.VMEM((SCR,), jnp.int32),
        pltpu.VMEM((SCR,), jnp.int32),
        pltpu.SemaphoreType.DMA,
    ],
)(_sc_count_body)


def _tc_tail_body(cnt_ref, hv_ref, tv_ref, rv_ref,
                  rq_ref, w_ref, b_ref, wm1_ref, bm1_ref, wm2_ref, bm2_ref,
                  out_ref):
    X = cnt_ref[...]
    row_i = jax.lax.broadcasted_iota(jnp.int32, (P, NSUB * P), 1)
    p_i = jax.lax.broadcasted_iota(jnp.int32, (P, NSUB * P), 0)
    sel = (lax.rem(row_i, P) == p_i).astype(jnp.float32)
    cnt_pl = sel @ X
    col_i = jax.lax.broadcasted_iota(jnp.int32, (R2 * L, R2), 0)
    r_i = jax.lax.broadcasted_iota(jnp.int32, (R2 * L, R2), 1)
    red = (lax.div(col_i, L) == r_i).astype(jnp.float32)
    cnt = cnt_pl @ red

    rq = rq_ref[...]
    S = cnt @ rq

    r_iota = jax.lax.broadcasted_iota(jnp.int32, (P, R2), 1)
    oh_r = (rv_ref[...] == r_iota).astype(jnp.float32)
    q = oh_r @ rq

    is_head = (tv_ref[...] == hv_ref[...]).astype(jnp.float32)
    agg = q * (S + is_head)
    hin = is_head * q
    h1 = jnp.maximum(hin @ w_ref[:DIM] + agg @ w_ref[DIM:] + b_ref[...], 0.0)
    hm = jnp.maximum(h1 @ wm1_ref[:DIM] + q @ wm1_ref[DIM:] + bm1_ref[...], 0.0)
    out_ref[...] = hm @ wm2_ref[...] + bm2_ref[...]


def kernel(edge_index, edge_type, rel_query, h_index, t_index, r_index,
           W, b, Wm1, bm1, Wm2, bm2):
    npad = E_PAD - E_TOTAL
    pad_i = jnp.full((npad,), -1, jnp.int32)
    pad_t = jnp.zeros((npad,), jnp.int32)
    src = jnp.concatenate([edge_index[0], pad_i])
    dst = jnp.concatenate([edge_index[1], pad_i])
    et = jnp.concatenate([edge_type, pad_t])
    hv = h_index.reshape(P)
    tv = t_index.reshape(P)
    pair = jnp.concatenate([jnp.repeat(h_index[:, 0], L),
                            jnp.repeat(tv, L)])

    cnt_all, _ = _sc_count(src, dst, et, pair)
    cnt2 = cnt_all.reshape(NSUB * P, R2 * L)

    whole = lambda shape: pl.BlockSpec(shape, lambda: tuple(0 for _ in shape))
    out = pl.pallas_call(
        _tc_tail_body,
        in_specs=[
            whole((NSUB * P, R2 * L)),
            whole((P, 1)), whole((P, 1)), whole((P, 1)),
            whole((R2, DIM)),
            whole((2 * DIM, DIM)), whole((1, DIM)),
            whole((2 * DIM, 2 * DIM)), whole((1, 2 * DIM)),
            whole((2 * DIM, 1)), whole((1, 1)),
        ],
        out_specs=whole((P, 1)),
        out_shape=jax.ShapeDtypeStruct((P, 1), jnp.float32),
    )(cnt2, hv.reshape(P, 1), tv.reshape(P, 1), r_index.reshape(P, 1),
      rel_query, W, b.reshape(1, DIM), Wm1, bm1.reshape(1, 2 * DIM),
      Wm2, bm2.reshape(1, 1))
    return out[:, 0].reshape(BS, NEG)


# speedup vs baseline: 1.1978x; 1.0146x over previous
import jax
import jax.numpy as jnp
from jax import lax
from jax.experimental import pallas as pl
from jax.experimental.pallas import tpu as pltpu
from jax.experimental.pallas import tpu_sc as plsc
import functools

N_NODES = 10000
NUM_REL = 16
DIM = 128
E_TOTAL = 160000
BS = 2
NEG = 8
P = BS * NEG
R2 = 2 * NUM_REL
L = 16
NSUB = 32
CH = 5120
E_PAD = NSUB * CH
NVEC = CH // L
HIST = P * R2 * L
G = 8
NGRP = NVEC // G
SCR = (NVEC + NGRP) * L


def _sc_count_body(src_hbm, dst_hbm, et_hbm, pair_hbm, out_hbm,
                   s_v, d_v, t_v, pair_v, cnt_v, scr_v, scr2_v, shr_v, sem):
    wid = lax.axis_index("s") * 2 + lax.axis_index("c")
    base = wid * CH
    c1 = pltpu.async_copy(src_hbm.at[pl.ds(base, CH)], s_v, sem)
    c2 = pltpu.async_copy(dst_hbm.at[pl.ds(base, CH)], d_v, sem)
    c3 = pltpu.async_copy(et_hbm.at[pl.ds(base, CH)], t_v, sem)
    c4 = pltpu.async_copy(pair_hbm, pair_v, sem)

    zv = jnp.zeros((L,), jnp.float32)

    def _zero(k, c):
        for u in range(8):
            cnt_v[pl.ds((k * 8 + u) * L, L)] = zv
        return c
    lax.fori_loop(0, HIST // (8 * L), _zero, 0)
    c1.wait()
    c2.wait()
    c3.wait()
    c4.wait()

    def hsld(b):
        return pair_v[pl.ds(b * L, L)]

    def tsld(p):
        return pair_v[pl.ds((BS + p) * L, L)]

    def _screen(g, c):
        off = g * G * L
        acc = jnp.zeros((L,), jnp.int32)
        for j in range(G):
            sv = s_v[pl.ds(off + j * L, L)]
            dv = d_v[pl.ds(off + j * L, L)]
            m_any = ((sv == hsld(0)) | (sv == hsld(1)) |
                     (dv == hsld(0)) | (dv == hsld(1)))
            w = jnp.where(m_any, jnp.int32(1), jnp.int32(0))
            scr_v[pl.ds(off + j * L, L)] = w
            acc = acc | w
        scr_v[pl.ds((NVEC + g) * L, L)] = acc
        return c
    lax.fori_loop(0, NGRP, _screen, 0)

    sid = lax.axis_index("s")
    pltpu.sync_copy(scr_v, shr_v.at[sid])
    pltpu.sync_copy(shr_v.at[sid], scr2_v)

    def _count_vec(off):
        sv = s_v[pl.ds(off, L)]
        dv = d_v[pl.ds(off, L)]
        tvv = t_v[pl.ds(off, L)]
        m1 = []
        m2 = []
        for b in range(BS):
            mf = sv == hsld(b)
            mi = dv == hsld(b)
            for j in range(NEG):
                p = b * NEG + j
                m1.append(jnp.logical_and(mf, dv == tsld(p)))
                m2.append(jnp.logical_and(mi, sv == tsld(p)))
        for r in range(NUM_REL):
            eq = tvv == r
            for p in range(P):
                v1 = jnp.where(jnp.logical_and(m1[p], eq), 1.0, 0.0)
                plsc.addupdate(cnt_v.at[pl.ds((p * R2 + r) * L, L)], v1)
                v2 = jnp.where(jnp.logical_and(m2[p], eq), 1.0, 0.0)
                plsc.addupdate(
                    cnt_v.at[pl.ds((p * R2 + NUM_REL + r) * L, L)], v2)

    def _step(g, c):
        goff = (NVEC + g) * L
        w = scr2_v[pl.ds(goff, L)]
        hit = w[0]
        for i in range(1, L):
            hit = hit | w[i]

        @pl.when(hit > 0)
        def _drill():
            def _dj(j, c2):
                off = (g * G + j) * L
                wv = scr2_v[pl.ds(off, L)]
                h2 = wv[0]
                for i in range(1, L):
                    h2 = h2 | wv[i]

                @pl.when(h2 > 0)
                def _count():
                    _count_vec(off)
                return c2
            lax.fori_loop(0, G, _dj, 0)
        return c
    lax.fori_loop(0, NGRP, _step, 0)

    pltpu.sync_copy(cnt_v, out_hbm.at[wid])


_sc_count = functools.partial(
    pl.kernel,
    out_type=jax.ShapeDtypeStruct((NSUB, HIST), jnp.float32),
    mesh=plsc.VectorSubcoreMesh(core_axis_name="c", subcore_axis_name="s"),
    scratch_types=[
        pltpu.VMEM((CH,), jnp.int32),
        pltpu.VMEM((CH,), jnp.int32),
        pltpu.VMEM((CH,), jnp.int32),
        pltpu.VMEM(((BS + P) * L,), jnp.int32),
        pltpu.VMEM((HIST,), jnp.float32),
        pltpu.VMEM((SCR,), jnp.int32),
        pltpu.VMEM((SCR,), jnp.int32),
        pltpu.VMEM_SHARED((16, SCR), jnp.int32),
        pltpu.SemaphoreType.DMA,
    ],
)(_sc_count_body)


def _tc_tail_body(cnt_ref, hv_ref, tv_ref, rv_ref,
                  rq_ref, w_ref, b_ref, wm1_ref, bm1_ref, wm2_ref, bm2_ref,
                  out_ref):
    X = cnt_ref[...]
    row_i = jax.lax.broadcasted_iota(jnp.int32, (P, NSUB * P), 1)
    p_i = jax.lax.broadcasted_iota(jnp.int32, (P, NSUB * P), 0)
    sel = (lax.rem(row_i, P) == p_i).astype(jnp.float32)
    cnt_pl = sel @ X
    col_i = jax.lax.broadcasted_iota(jnp.int32, (R2 * L, R2), 0)
    r_i = jax.lax.broadcasted_iota(jnp.int32, (R2 * L, R2), 1)
    red = (lax.div(col_i, L) == r_i).astype(jnp.float32)
    cnt = cnt_pl @ red

    rq = rq_ref[...]
    S = cnt @ rq

    r_iota = jax.lax.broadcasted_iota(jnp.int32, (P, R2), 1)
    oh_r = (rv_ref[...] == r_iota).astype(jnp.float32)
    q = oh_r @ rq

    is_head = (tv_ref[...] == hv_ref[...]).astype(jnp.float32)
    agg = q * (S + is_head)
    hin = is_head * q
    h1 = jnp.maximum(hin @ w_ref[:DIM] + agg @ w_ref[DIM:] + b_ref[...], 0.0)
    hm = jnp.maximum(h1 @ wm1_ref[:DIM] + q @ wm1_ref[DIM:] + bm1_ref[...], 0.0)
    out_ref[...] = hm @ wm2_ref[...] + bm2_ref[...]


def kernel(edge_index, edge_type, rel_query, h_index, t_index, r_index,
           W, b, Wm1, bm1, Wm2, bm2):
    npad = E_PAD - E_TOTAL
    pad_i = jnp.full((npad,), -1, jnp.int32)
    pad_t = jnp.zeros((npad,), jnp.int32)
    src = jnp.concatenate([edge_index[0], pad_i])
    dst = jnp.concatenate([edge_index[1], pad_i])
    et = jnp.concatenate([edge_type, pad_t])
    hv = h_index.reshape(P)
    tv = t_index.reshape(P)
    pair = jnp.concatenate([jnp.repeat(h_index[:, 0], L),
                            jnp.repeat(tv, L)])

    cnt_all = _sc_count(src, dst, et, pair)
    cnt2 = cnt_all.reshape(NSUB * P, R2 * L)

    whole = lambda shape: pl.BlockSpec(shape, lambda: tuple(0 for _ in shape))
    out = pl.pallas_call(
        _tc_tail_body,
        in_specs=[
            whole((NSUB * P, R2 * L)),
            whole((P, 1)), whole((P, 1)), whole((P, 1)),
            whole((R2, DIM)),
            whole((2 * DIM, DIM)), whole((1, DIM)),
            whole((2 * DIM, 2 * DIM)), whole((1, 2 * DIM)),
            whole((2 * DIM, 1)), whole((1, 1)),
        ],
        out_specs=whole((P, 1)),
        out_shape=jax.ShapeDtypeStruct((P, 1), jnp.float32),
    )(cnt2, hv.reshape(P, 1), tv.reshape(P, 1), r_index.reshape(P, 1),
      rel_query, W, b.reshape(1, DIM), Wm1, bm1.reshape(1, 2 * DIM),
      Wm2, bm2.reshape(1, 1))
    return out[:, 0].reshape(BS, NEG)
